# baseline (speedup 1.0000x reference)
import jax
import jax.numpy as jnp
from jax import lax
from jax.experimental import pallas as pl
from jax.experimental.pallas import tpu as pltpu
from jax.experimental.pallas import tpu_sc as plsc

D_MODEL = 128
NC = 2
NS = 16
NW = NC * NS

G = 128
NCHUNK = 200
NB = 4
AHEAD = NB - 1
RB = 2

PERIOD = 16
DIRECT = (7, 15)
_spmem_idx = {}
_n = 0
for _p in range(PERIOD):
    if _p not in DIRECT:
        _spmem_idx[_p] = _n % RB
        _n += 1
MACROS = NCHUNK // PERIOD
TAIL = NCHUNK - MACROS * PERIOD


def _emb_body(x_hbm, table_hbm, out_hbm, idx_v, shr, *scratch):
    rows = scratch[:NB]
    wsem = scratch[NB:2 * NB]
    dsem = scratch[2 * NB:2 * NB + RB]
    gsem = scratch[2 * NB + RB:]
    cc = lax.axis_index("c")
    ss = lax.axis_index("s")
    wid = ss * NC + cc
    pltpu.sync_copy(x_hbm.at[wid], idx_v)
    base = wid * (NCHUNK * G)

    def fire_g(ci, b):
        pltpu.async_copy(table_hbm.at[idx_v.at[ci]], rows[b], gsem[b])

    def wait_g(b):
        pltpu.make_async_copy(table_hbm.at[pl.ds(0, G)], rows[b], gsem[b]).wait()

    def fire_x(b, r):
        pltpu.async_copy(rows[b], shr.at[ss, r], wsem[b])

    def fire_direct(ci, b):
        pltpu.async_copy(rows[b], out_hbm.at[pl.ds(base + ci * G, G)], wsem[b])

    def wait_x(b):
        pltpu.make_async_copy(rows[b], shr.at[ss, 0], wsem[b]).wait()

    def fire_d(ci, r):
        pltpu.async_copy(shr.at[ss, r], out_hbm.at[pl.ds(base + ci * G, G)],
                         dsem[r])

    def wait_d(r):
        pltpu.make_async_copy(shr.at[ss, r], out_hbm.at[pl.ds(base, G)],
                              dsem[r]).wait()

    for b in range(AHEAD):
        fire_g(b, b)

    def slot(ci, pos, first_sp=(), first=False, may_fire=True):
        b = pos % NB
        prev = (pos - 1) % PERIOD
        wait_g(b)
        if not (first and pos == 0):
            wait_x((b + NB - 1) % NB)
            if prev not in DIRECT:
                fire_d(ci - 1, _spmem_idx[prev])
        if pos in DIRECT:
            fire_direct(ci, b)
        else:
            r = _spmem_idx[pos]
            if not (first and pos in first_sp):
                wait_d(r)
            fire_x(b, r)
        if may_fire:
            fire_g(ci + AHEAD, (b + NB - 1) % NB)

    for pos in range(PERIOD):
        slot(pos, pos, first_sp=(0, 1), first=True)

    def step(s, carry):
        for pos in range(PERIOD):
            slot(s * PERIOD + pos, pos)
        return carry

    lax.fori_loop(1, MACROS, step, 0)

    for pos in range(TAIL):
        ci = MACROS * PERIOD + pos
        slot(ci, pos, may_fire=(ci + AHEAD < NCHUNK))

    wait_x((NCHUNK - 1) % NB)
    wait_d(1)
    wait_d(0)


@jax.jit
def _emb(xf, table):
    mesh = plsc.VectorSubcoreMesh(core_axis_name="c", subcore_axis_name="s")
    kern = pl.kernel(
        _emb_body,
        out_type=jax.ShapeDtypeStruct((NW * NCHUNK * G, D_MODEL), jnp.float32),
        mesh=mesh,
        scratch_types=(
            [pltpu.VMEM((NCHUNK, G), jnp.int32),
             pltpu.VMEM_SHARED((NS, RB, G, D_MODEL), jnp.float32)]
            + [pltpu.VMEM((G, D_MODEL), jnp.float32) for _ in range(NB)]
            + [pltpu.SemaphoreType.DMA for _ in range(NB + RB + NB)]
        ),
    )
    return kern(xf, table)


def kernel(x, table):
    b, t = x.shape
    xf = x.reshape(NW, NCHUNK, G).astype(jnp.int32)
    out = _emb(xf, table)
    return out.reshape(b, t, D_MODEL)
